# Initial kernel scaffold; baseline (speedup 1.0000x reference)
import jax
import jax.numpy as jnp
from jax.experimental import pallas as pl


def kernel(boxes, scores):
    raise NotImplementedError("write your pallas kernel here")


# speedup vs baseline: 14.1733x; 14.1733x over previous
import functools

import jax
import jax.numpy as jnp
from jax import lax
from jax.experimental import pallas as pl
from jax.experimental.pallas import tpu as pltpu

_IOU_THR = 0.6
_MAX_OUT = 100
_ROWS = 160
_LANES = 128
_NP = _ROWS * _LANES


def _nms_body(n_real, bt_ref, st_ref, box_o, sc_o, lb_o):
    x1 = bt_ref[0]
    y1 = bt_ref[1]
    x2 = bt_ref[2]
    y2 = bt_ref[3]
    areas = (x2 - x1) * (y2 - y1)

    num_classes = st_ref.shape[0]

    def class_body(c, carry):
        m, lab = carry
        s = st_ref[c]
        gt = s > m
        m = jnp.where(gt, s, m)
        lab = jnp.where(gt, c, lab)
        return m, lab

    smax, labels = lax.fori_loop(
        1, num_classes, class_body,
        (st_ref[0], jnp.zeros((_ROWS, _LANES), jnp.int32)))

    rowi = lax.broadcasted_iota(jnp.int32, (_ROWS, _LANES), 0)
    coli = lax.broadcasted_iota(jnp.int32, (_ROWS, _LANES), 1)
    iota = rowi * _LANES + coli
    masked0 = jnp.where(iota < n_real, smax, -1.0)

    def nms_iter(i, masked):
        m = jnp.max(masked)
        idx = jnp.min(jnp.where(masked == m, iota, jnp.int32(1 << 30)))
        any_alive = m >= 0.0
        eq = iota == idx
        eqf = eq.astype(jnp.float32)
        bx1 = jnp.sum(x1 * eqf)
        by1 = jnp.sum(y1 * eqf)
        bx2 = jnp.sum(x2 * eqf)
        by2 = jnp.sum(y2 * eqf)
        sc = jnp.sum(smax * eqf)
        lb = jnp.sum(jnp.where(eq, labels, 0))
        box_o[i, 0] = bx1
        box_o[i, 1] = by1
        box_o[i, 2] = bx2
        box_o[i, 3] = by2
        sc_o[i] = sc
        lb_o[i] = lb
        xx1 = jnp.maximum(x1, bx1)
        yy1 = jnp.maximum(y1, by1)
        xx2 = jnp.minimum(x2, bx2)
        yy2 = jnp.minimum(y2, by2)
        w = jnp.maximum(xx2 - xx1, 0.0)
        h = jnp.maximum(yy2 - yy1, 0.0)
        inter = w * h
        area_b = (bx2 - bx1) * (by2 - by1)
        iou = inter / (areas + area_b - inter + 1e-9)
        keep = (iou <= _IOU_THR) & any_alive & jnp.logical_not(eq)
        return jnp.where(keep, masked, -1.0)

    lax.fori_loop(0, _MAX_OUT, nms_iter, masked0)


def kernel(boxes, scores):
    n = boxes.shape[1]
    num_classes = scores.shape[2]
    boxes_f = boxes.reshape(n, 4)
    scores_f = scores.reshape(n, num_classes)
    pad = _NP - n
    boxes_p = jnp.pad(boxes_f, ((0, pad), (0, 0)))
    scores_p = jnp.pad(scores_f, ((0, pad), (0, 0)))
    bt = boxes_p.T.reshape(4, _ROWS, _LANES)
    st = scores_p.T.reshape(num_classes, _ROWS, _LANES)

    box_o, sc_o, lb_o = pl.pallas_call(
        functools.partial(_nms_body, n),
        out_shape=(
            jax.ShapeDtypeStruct((_MAX_OUT, 4), jnp.float32),
            jax.ShapeDtypeStruct((_MAX_OUT,), jnp.float32),
            jax.ShapeDtypeStruct((_MAX_OUT,), jnp.int32),
        ),
        out_specs=(
            pl.BlockSpec(memory_space=pltpu.SMEM),
            pl.BlockSpec(memory_space=pltpu.SMEM),
            pl.BlockSpec(memory_space=pltpu.SMEM),
        ),
    )(bt, st)
    return box_o[None], sc_o[None], lb_o[None]
